# baseline (speedup 1.0000x reference)
import dataclasses
import functools

import jax
import jax.numpy as jnp
from jax import lax
from jax.experimental import pallas as pl
from jax.experimental.pallas import tpu as pltpu
from jax.experimental.pallas import tpu_sc as plsc

N = 10000
E = 160000
L = 256
D = 256
DH = D // 2

_NW = 32
_EPW = 5120
_EPAD = _NW * _EPW
_CA = 64
_NCA = _EPW // _CA
_CB = 64
_EPT_B = _EPAD // 16
_NCB = _EPT_B // _CB

_ROWS = 1000

_mesh = plsc.VectorSubcoreMesh(core_axis_name="c", subcore_axis_name="s")

_sc_params = pltpu.CompilerParams()
if "needs_layout_passes" in pltpu.CompilerParams.__dataclass_fields__:
    _sc_params = dataclasses.replace(_sc_params, needs_layout_passes=False)


def _proj_body(x_ref, w_ref, b_ref, o_ref):
    acc = jnp.dot(x_ref[...], w_ref[...],
                  preferred_element_type=jnp.float32,
                  precision=jax.lax.Precision.HIGHEST) + b_ref[...]
    o_ref[...] = acc


def _vproj_body(x_ref, w_ref, b_ref, o_ref, s_ref):
    acc = jnp.dot(x_ref[...], w_ref[...],
                  preferred_element_type=jnp.float32,
                  precision=jax.lax.Precision.HIGHEST) + b_ref[...]
    o_ref[0, :, :] = acc[:, 0:DH]
    o_ref[1, :, :] = acc[:, DH:D]
    s_ref[...] = acc[:, D:2 * D]


def _projections(x, w2, b2c):
    return pl.pallas_call(
        _proj_body,
        grid=(N // _ROWS,),
        in_specs=[
            pl.BlockSpec((_ROWS, L), lambda i: (i, 0)),
            pl.BlockSpec((L, 2 * D), lambda i: (0, 0)),
            pl.BlockSpec((1, 2 * D), lambda i: (0, 0)),
        ],
        out_specs=pl.BlockSpec((_ROWS, 2 * D), lambda i: (i, 0)),
        out_shape=jax.ShapeDtypeStruct((N, 2 * D), jnp.float32),
    )(x, w2, b2c)


def _vprojections(x, w2, b2c):
    return pl.pallas_call(
        _vproj_body,
        grid=(N // _ROWS,),
        in_specs=[
            pl.BlockSpec((_ROWS, L), lambda i: (i, 0)),
            pl.BlockSpec((L, 2 * D), lambda i: (0, 0)),
            pl.BlockSpec((1, 2 * D), lambda i: (0, 0)),
        ],
        out_specs=[pl.BlockSpec((2, _ROWS, DH), lambda i: (0, i, 0)),
                   pl.BlockSpec((_ROWS, D), lambda i: (i, 0))],
        out_shape=[jax.ShapeDtypeStruct((2, N, DH), jnp.float32),
                   jax.ShapeDtypeStruct((N, D), jnp.float32)],
    )(x, w2, b2c)


@functools.partial(
    pl.kernel,
    mesh=_mesh,
    compiler_params=_sc_params,
    out_type=(
        jax.ShapeDtypeStruct((_EPAD,), jnp.float32),
        jax.ShapeDtypeStruct((2, N), jnp.float32),
    ),
    scratch_types=[
        pltpu.VMEM((_NCA, _CA), jnp.int32),
        pltpu.VMEM((_NCA, _CA), jnp.int32),
        pltpu.VMEM((2, _CA, D), jnp.float32),
        pltpu.VMEM((2, _CA, D), jnp.float32),
        pltpu.VMEM((16, 16), jnp.float32),
        pltpu.VMEM((4, _CA), jnp.float32),
        pltpu.VMEM_SHARED((N,), jnp.float32),
        pltpu.SemaphoreType.DMA,
        pltpu.SemaphoreType.DMA,
        pltpu.SemaphoreType.DMA,
        pltpu.SemaphoreType.DMA,
        pltpu.SemaphoreType.DMA,
        pltpu.SemaphoreType.DMA,
        pltpu.SemaphoreType.DMA,
        pltpu.SemaphoreType.DMA,
        pltpu.SemaphoreType.DMA,
        pltpu.SemaphoreType.DMA,
        pltpu.SemaphoreType.DMA,
        pltpu.SemaphoreType.DMA,
    ],
)
def _edge_softmax(q_hbm, k_hbm, src3_hbm, dst3_hbm, zn_hbm, ex_hbm, den_hbm,
                  src_all, dst_all, qd, ks, accb, exv, den_sh,
                  gq0, gq1, gk0, gk1,
                  ws0, ws1, ws2, ws3, wd0, wd1, wd2, wd3):
    cid = lax.axis_index("c")
    sid = lax.axis_index("s")
    wid = sid * 2 + (1 - cid)
    base = wid * _EPW
    lanes = lax.iota(jnp.int32, 16)
    gq = (gq0, gq1)
    gk = (gk0, gk1)
    ws = (ws0, ws1, ws2, ws3)
    wd = (wd0, wd1, wd2, wd3)

    pltpu.sync_copy(src3_hbm.at[wid], src_all)
    pltpu.sync_copy(dst3_hbm.at[wid], dst_all)

    @pl.when(sid == 0)
    def _():
        pltpu.sync_copy(zn_hbm, den_sh)

    plsc.subcore_barrier()

    def issue_gather(c, b):
        pltpu.async_copy(q_hbm.at[dst_all.at[c]], qd.at[b], gq[b])
        pltpu.async_copy(k_hbm.at[src_all.at[c]], ks.at[b], gk[b])

    def wait_gather(c, b):
        pltpu.make_async_copy(q_hbm.at[dst_all.at[c]], qd.at[b], gq[b]).wait()
        pltpu.make_async_copy(k_hbm.at[src_all.at[c]], ks.at[b], gk[b]).wait()

    def issue_writes(c, b4):
        off = base + c * _CA
        pltpu.async_copy(exv.at[b4], ex_hbm.at[pl.ds(off, _CA)], ws[b4])
        pltpu.async_copy(exv.at[b4], den_sh.at[dst_all.at[c]], wd[b4],
                         add=True)

    def wait_writes(c, b4):
        off = base + c * _CA
        pltpu.make_async_copy(exv.at[b4], ex_hbm.at[pl.ds(off, _CA)],
                              ws[b4]).wait()
        pltpu.make_async_copy(exv.at[b4], den_sh.at[dst_all.at[c]],
                              wd[b4]).wait()

    issue_gather(0, 0)
    issue_gather(1, 1)

    @pl.loop(0, _NCA, step=4)
    def _(c0):
        for db in range(4):
            c = c0 + db
            b2 = db % 2
            b4 = db
            wait_gather(c, b2)

            @pl.loop(0, _CA, step=16)
            def _(g):
                for e in range(16):
                    acc = (qd[b2, g + e, pl.ds(0, 16)]
                           * ks[b2, g + e, pl.ds(0, 16)])
                    for j in range(1, 16):
                        acc = acc + (qd[b2, g + e, pl.ds(16 * j, 16)]
                                     * ks[b2, g + e, pl.ds(16 * j, 16)])
                    accb[e, :] = acc
                tot = plsc.load_gather(accb, [lanes, jnp.zeros(16, jnp.int32)])
                for j in range(1, 16):
                    tot = tot + plsc.load_gather(
                        accb, [lanes, jnp.full(16, j, jnp.int32)])
                a16 = jnp.exp(tot * (1.0 / 16.0))
                a16 = jnp.where(base + c * _CA + g + lanes < E, a16, 0.0)
                exv[b4, pl.ds(g, 16)] = a16

            issue_writes(c, b4)

            @pl.when(c >= 2)
            def _():
                wait_writes(c - 2, (db + 2) % 4)

            @pl.when(c + 2 < _NCA)
            def _():
                issue_gather(c + 2, b2)

    wait_writes(_NCA - 2, (_NCA - 2) % 4)
    wait_writes(_NCA - 1, (_NCA - 1) % 4)

    plsc.subcore_barrier()

    @pl.when(sid == 0)
    def _():
        pltpu.sync_copy(den_sh, den_hbm.at[cid])


@functools.partial(
    pl.kernel,
    mesh=_mesh,
    compiler_params=_sc_params,
    out_type=jax.ShapeDtypeStruct((2, N, DH), jnp.float32),
    scratch_types=[
        pltpu.VMEM((8, _CB), jnp.int32),
        pltpu.VMEM((8, _CB), jnp.int32),
        pltpu.VMEM((8, _CB), jnp.float32),
        pltpu.VMEM((4, _CB, DH), jnp.float32),
        pltpu.VMEM_SHARED((N, DH), jnp.float32),
        pltpu.SemaphoreType.DMA,
        pltpu.SemaphoreType.DMA,
        pltpu.SemaphoreType.DMA,
        pltpu.SemaphoreType.DMA,
        pltpu.SemaphoreType.DMA,
        pltpu.SemaphoreType.DMA,
        pltpu.SemaphoreType.DMA,
        pltpu.SemaphoreType.DMA,
        pltpu.SemaphoreType.DMA,
        pltpu.SemaphoreType.DMA,
        pltpu.SemaphoreType.DMA,
        pltpu.SemaphoreType.DMA,
        pltpu.SemaphoreType.DMA,
        pltpu.SemaphoreType.DMA,
        pltpu.SemaphoreType.DMA,
        pltpu.SemaphoreType.DMA,
    ],
)
def _edge_aggregate(vcat_hbm, src2_hbm, dstb_hbm, exb_hbm, znd_hbm, out_hbm,
                    srcc, dstc, exc, rows, out_sh,
                    gs0, gs1, gs2, gs3, ss0, ss1, ss2, ss3,
                    sm0, sm1, sm2, sm3, sm4, sm5, sm6, sm7):
    cid = lax.axis_index("c")
    sid = lax.axis_index("s")
    gs = (gs0, gs1, gs2, gs3)
    ss = (ss0, ss1, ss2, ss3)
    sm = (sm0, sm1, sm2, sm3, sm4, sm5, sm6, sm7)

    @pl.when(sid == 0)
    def _():
        pltpu.sync_copy(znd_hbm, out_sh)

    plsc.subcore_barrier()

    def issue_small(c, b8):
        pltpu.async_copy(src2_hbm.at[cid, sid, c], srcc.at[b8], sm[b8])
        pltpu.async_copy(dstb_hbm.at[sid, c], dstc.at[b8], sm[b8])
        pltpu.async_copy(exb_hbm.at[sid, c], exc.at[b8], sm[b8])

    def wait_small(c, b8):
        pltpu.make_async_copy(src2_hbm.at[cid, sid, c], srcc.at[b8],
                              sm[b8]).wait()
        pltpu.make_async_copy(dstb_hbm.at[sid, c], dstc.at[b8],
                              sm[b8]).wait()
        pltpu.make_async_copy(exb_hbm.at[sid, c], exc.at[b8],
                              sm[b8]).wait()

    def issue_gather(c, b4, b8):
        pltpu.async_copy(vcat_hbm.at[srcc.at[b8]], rows.at[b4], gs[b4])

    def wait_gather(c, b4, b8):
        pltpu.make_async_copy(vcat_hbm.at[srcc.at[b8]], rows.at[b4],
                              gs[b4]).wait()

    def issue_scatter(c, b4, b8):
        pltpu.async_copy(rows.at[b4], out_sh.at[dstc.at[b8]], ss[b4],
                         add=True)

    def wait_scatter(c, b4, b8):
        pltpu.make_async_copy(rows.at[b4], out_sh.at[dstc.at[b8]],
                              ss[b4]).wait()

    for c in range(4):
        issue_small(c, c)
    for c in range(2):
        wait_small(c, c)
        issue_gather(c, c, c)

    @pl.loop(0, _NCB, step=8)
    def _(c0):
        for db in range(8):
            c = c0 + db
            b4 = db % 4
            b8 = db
            wait_gather(c, b4, b8)

            @pl.loop(0, _CB, step=2)
            def _(e):
                for de in range(2):
                    s = plsc.load_gather(
                        exc, [jnp.full(16, b8, jnp.int32),
                              jnp.full(16, e + de, jnp.int32)])
                    for j in range(DH // 16):
                        sl = pl.ds(16 * j, 16)
                        rows[b4, e + de, sl] = rows[b4, e + de, sl] * s

            issue_scatter(c, b4, b8)

            @pl.when(c >= 2)
            def _():
                wait_scatter(c - 2, (db + 2) % 4, (db + 6) % 8)

            @pl.when(c + 2 < _NCB)
            def _():
                wait_small(c + 2, (db + 2) % 8)
                issue_gather(c + 2, (db + 2) % 4, (db + 2) % 8)

            @pl.when(c + 4 < _NCB)
            def _():
                issue_small(c + 4, (db + 4) % 8)

    wait_scatter(_NCB - 2, (_NCB - 2) % 4, (_NCB - 2) % 8)
    wait_scatter(_NCB - 1, (_NCB - 1) % 4, (_NCB - 1) % 8)

    plsc.subcore_barrier()

    @pl.when(sid == 0)
    def _():
        pltpu.sync_copy(out_sh, out_hbm.at[cid])


_NB = 10


def _epilogue_body(outp_ref, den_ref, skip_ref, wkp_ref, bkp_ref,
                   wvp_ref, bvp_ref, qv_ref, wop_ref, bop_ref,
                   h_ref, s_ref, y_ref, m_ref, l_ref,
                   m_s, l_s, num_s):
    i = pl.program_id(0)
    out = jnp.concatenate([outp_ref[0], outp_ref[1]], axis=1)
    den = jnp.maximum(den_ref[0] + den_ref[1], 1e-16)
    z = out / den + skip_ref[...]
    hb = 0.5 * z * (1.0 + lax.erf(z * 0.7071067811865476))
    h_ref[...] = hb

    kp = jnp.dot(hb, wkp_ref[...], preferred_element_type=jnp.float32,
                 precision=jax.lax.Precision.HIGHEST) + bkp_ref[...]
    sb = jnp.dot(kp, qv_ref[...].T, preferred_element_type=jnp.float32,
                 precision=jax.lax.Precision.HIGHEST) * (1.0 / 16.0)
    s_ref[...] = sb
    vp = jnp.dot(hb, wvp_ref[...], preferred_element_type=jnp.float32,
                 precision=jax.lax.Precision.HIGHEST) + bvp_ref[...]

    @pl.when(i == 0)
    def _():
        m_s[0, 0] = -jnp.inf
        l_s[0, 0] = 0.0
        num_s[...] = jnp.zeros_like(num_s)

    m_old = m_s[0, 0]
    m_new = jnp.maximum(m_old, jnp.max(sb))
    corr = jnp.exp(m_old - m_new)
    w = jnp.exp(sb - m_new)
    l_s[0, 0] = l_s[0, 0] * corr + jnp.sum(w)
    num_s[...] = num_s[...] * corr + jnp.dot(
        w.T, vp, preferred_element_type=jnp.float32,
        precision=jax.lax.Precision.HIGHEST)
    m_s[0, 0] = m_new

    @pl.when(i == _NB - 1)
    def _():
        pooled = num_s[...] / l_s[0, 0]
        y_ref[...] = jnp.dot(pooled, wop_ref[...],
                             preferred_element_type=jnp.float32,
                             precision=jax.lax.Precision.HIGHEST) + bop_ref[...]
        m_ref[...] = jnp.full((1, 1), m_s[0, 0], jnp.float32)
        l_ref[...] = jnp.full((1, 1), l_s[0, 0], jnp.float32)


def _epilogue(outp, den2, skip, wkp, bkp, wvp, bvp, qv, wop, bop):
    r = N // _NB
    return pl.pallas_call(
        _epilogue_body,
        grid=(_NB,),
        in_specs=[
            pl.BlockSpec((2, r, DH), lambda i: (0, i, 0)),
            pl.BlockSpec((2, r, 1), lambda i: (0, i, 0)),
            pl.BlockSpec((r, D), lambda i: (i, 0)),
            pl.BlockSpec((D, D), lambda i: (0, 0)),
            pl.BlockSpec((1, D), lambda i: (0, 0)),
            pl.BlockSpec((D, D), lambda i: (0, 0)),
            pl.BlockSpec((1, D), lambda i: (0, 0)),
            pl.BlockSpec((1, D), lambda i: (0, 0)),
            pl.BlockSpec((D, D), lambda i: (0, 0)),
            pl.BlockSpec((1, D), lambda i: (0, 0)),
        ],
        out_specs=[
            pl.BlockSpec((r, D), lambda i: (i, 0)),
            pl.BlockSpec((r, 1), lambda i: (i, 0)),
            pl.BlockSpec((1, D), lambda i: (0, 0)),
            pl.BlockSpec((1, 1), lambda i: (0, 0)),
            pl.BlockSpec((1, 1), lambda i: (0, 0)),
        ],
        out_shape=[
            jax.ShapeDtypeStruct((N, D), jnp.float32),
            jax.ShapeDtypeStruct((N, 1), jnp.float32),
            jax.ShapeDtypeStruct((1, D), jnp.float32),
            jax.ShapeDtypeStruct((1, 1), jnp.float32),
            jax.ShapeDtypeStruct((1, 1), jnp.float32),
        ],
        scratch_shapes=[
            pltpu.SMEM((1, 1), jnp.float32),
            pltpu.SMEM((1, 1), jnp.float32),
            pltpu.VMEM((1, D), jnp.float32),
        ],
    )(outp, den2, skip, wkp, bkp, wvp, bvp, qv, wop, bop)


def _attn_body(s_ref, m_ref, l_ref, a_ref):
    a_ref[...] = jnp.exp(s_ref[...] - m_ref[0, 0]) / l_ref[0, 0]


def _attn_finalize(scores, m, l):
    r = N // _NB
    return pl.pallas_call(
        _attn_body,
        grid=(_NB,),
        in_specs=[
            pl.BlockSpec((r, 1), lambda i: (i, 0)),
            pl.BlockSpec((1, 1), lambda i: (0, 0)),
            pl.BlockSpec((1, 1), lambda i: (0, 0)),
        ],
        out_specs=pl.BlockSpec((r, 1), lambda i: (i, 0)),
        out_shape=jax.ShapeDtypeStruct((N, 1), jnp.float32),
    )(scores, m, l)


def kernel(x, edge_index, label, params):
    p = params
    wqk = jnp.concatenate([p['Wq'], p['Wk']], axis=1)
    bqk = jnp.concatenate([p['bq'], p['bk']])[None, :]
    wvs = jnp.concatenate([p['Wv'], p['Ws']], axis=1)
    bvs = jnp.concatenate([p['bv'], p['bs']])[None, :]
    qk = _projections(x, wqk, bqk)
    q = qk[:, 0:D]
    k = qk[:, D:2 * D]
    v2, skip = _vprojections(x, wvs, bvs)
    vcat = v2.reshape(2 * N, DH)

    src = jnp.pad(edge_index[0], (0, _EPAD - E))
    dst = jnp.pad(edge_index[1], (0, _EPAD - E))
    src3 = src.reshape(_NW, _NCA, _CA)
    dst3 = dst.reshape(_NW, _NCA, _CA)
    srcb = src.reshape(16, _NCB, _CB)
    dstb = dst.reshape(16, _NCB, _CB)
    src2 = jnp.stack([srcb, srcb + N])
    zn = jnp.zeros((N,), jnp.float32)
    znd = jnp.zeros((N, DH), jnp.float32)

    ex, den2 = _edge_softmax(q, k, src3, dst3, zn)
    exb = ex.reshape(16, _NCB, _CB)
    outp = _edge_aggregate(vcat, src2, dstb, exb, znd)

    qv = p['seed'] @ p['Wqp'] + p['bqp']
    h, scores, y, m, l = _epilogue(
        outp, den2[:, :, None], skip, p['Wkp'], p['bkp'][None, :],
        p['Wvp'], p['bvp'][None, :], qv, p['Wop'], p['bop'][None, :])
    A = _attn_finalize(scores, m, l)[:, 0]
    return (y, A, h)


# speedup vs baseline: 1.0767x; 1.0767x over previous
import dataclasses
import functools

import jax
import jax.numpy as jnp
from jax import lax
from jax.experimental import pallas as pl
from jax.experimental.pallas import tpu as pltpu
from jax.experimental.pallas import tpu_sc as plsc

N = 10000
E = 160000
L = 256
D = 256
DH = D // 2

_NW = 32
_EPW = 5120
_EPAD = _NW * _EPW
_CA = 64
_NCA = _EPW // _CA
_NCA0 = 104
_NCA1 = 56
_CB = 64
_EPT_B = _EPAD // 16
_NCB = _EPT_B // _CB

_ROWS = 1000

_mesh = plsc.VectorSubcoreMesh(core_axis_name="c", subcore_axis_name="s")

_sc_params = pltpu.CompilerParams()
if "needs_layout_passes" in pltpu.CompilerParams.__dataclass_fields__:
    _sc_params = dataclasses.replace(_sc_params, needs_layout_passes=False)


def _proj_body(x_ref, w_ref, b_ref, o_ref):
    acc = jnp.dot(x_ref[...], w_ref[...],
                  preferred_element_type=jnp.float32,
                  precision=jax.lax.Precision.HIGHEST) + b_ref[...]
    o_ref[...] = acc


def _vproj_body(x_ref, w_ref, b_ref, o_ref, s_ref):
    acc = jnp.dot(x_ref[...], w_ref[...],
                  preferred_element_type=jnp.float32,
                  precision=jax.lax.Precision.HIGHEST) + b_ref[...]
    o_ref[0, :, :] = acc[:, 0:DH]
    o_ref[1, :, :] = acc[:, DH:D]
    s_ref[...] = acc[:, D:2 * D]


def _projections(x, w2, b2c):
    return pl.pallas_call(
        _proj_body,
        grid=(N // _ROWS,),
        in_specs=[
            pl.BlockSpec((_ROWS, L), lambda i: (i, 0)),
            pl.BlockSpec((L, 2 * D), lambda i: (0, 0)),
            pl.BlockSpec((1, 2 * D), lambda i: (0, 0)),
        ],
        out_specs=pl.BlockSpec((_ROWS, 2 * D), lambda i: (i, 0)),
        out_shape=jax.ShapeDtypeStruct((N, 2 * D), jnp.float32),
    )(x, w2, b2c)


def _vprojections(x, w2, b2c):
    return pl.pallas_call(
        _vproj_body,
        grid=(N // _ROWS,),
        in_specs=[
            pl.BlockSpec((_ROWS, L), lambda i: (i, 0)),
            pl.BlockSpec((L, 2 * D), lambda i: (0, 0)),
            pl.BlockSpec((1, 2 * D), lambda i: (0, 0)),
        ],
        out_specs=[pl.BlockSpec((2, _ROWS, DH), lambda i: (0, i, 0)),
                   pl.BlockSpec((_ROWS, D), lambda i: (i, 0))],
        out_shape=[jax.ShapeDtypeStruct((2, N, DH), jnp.float32),
                   jax.ShapeDtypeStruct((N, D), jnp.float32)],
    )(x, w2, b2c)


@functools.partial(
    pl.kernel,
    mesh=_mesh,
    compiler_params=_sc_params,
    out_type=(
        jax.ShapeDtypeStruct((_EPAD,), jnp.float32),
        jax.ShapeDtypeStruct((2, N), jnp.float32),
    ),
    scratch_types=[
        pltpu.VMEM((_NCA0, _CA), jnp.int32),
        pltpu.VMEM((_NCA0, _CA), jnp.int32),
        pltpu.VMEM((2, _CA, D), jnp.float32),
        pltpu.VMEM((2, _CA, D), jnp.float32),
        pltpu.VMEM((16, 16), jnp.float32),
        pltpu.VMEM((4, _CA), jnp.float32),
        pltpu.VMEM_SHARED((N,), jnp.float32),
        pltpu.SemaphoreType.DMA,
        pltpu.SemaphoreType.DMA,
        pltpu.SemaphoreType.DMA,
        pltpu.SemaphoreType.DMA,
        pltpu.SemaphoreType.DMA,
        pltpu.SemaphoreType.DMA,
        pltpu.SemaphoreType.DMA,
        pltpu.SemaphoreType.DMA,
        pltpu.SemaphoreType.DMA,
        pltpu.SemaphoreType.DMA,
        pltpu.SemaphoreType.DMA,
        pltpu.SemaphoreType.DMA,
    ],
)
def _edge_softmax(q_hbm, k_hbm, src3_hbm, dst3_hbm, zn_hbm, ex_hbm, den_hbm,
                  src_all, dst_all, qd, ks, accb, exv, den_sh,
                  gq0, gq1, gk0, gk1,
                  ws0, ws1, ws2, ws3, wd0, wd1, wd2, wd3):
    cid = lax.axis_index("c")
    sid = lax.axis_index("s")
    nca = jnp.where(cid == 0, _NCA0, _NCA1)
    cbase = pl.multiple_of(sid * (_NCA0 + _NCA1) + cid * _NCA0, 8)
    base = cbase * _CA
    lanes = lax.iota(jnp.int32, 16)
    gq = (gq0, gq1)
    gk = (gk0, gk1)
    ws = (ws0, ws1, ws2, ws3)
    wd = (wd0, wd1, wd2, wd3)

    @pl.when(cid == 0)
    def _():
        pltpu.sync_copy(src3_hbm.at[pl.ds(cbase, _NCA0)], src_all)
        pltpu.sync_copy(dst3_hbm.at[pl.ds(cbase, _NCA0)], dst_all)

    @pl.when(cid == 1)
    def _():
        pltpu.sync_copy(src3_hbm.at[pl.ds(cbase, _NCA1)],
                        src_all.at[pl.ds(0, _NCA1)])
        pltpu.sync_copy(dst3_hbm.at[pl.ds(cbase, _NCA1)],
                        dst_all.at[pl.ds(0, _NCA1)])

    @pl.when(sid == 0)
    def _():
        pltpu.sync_copy(zn_hbm, den_sh)

    plsc.subcore_barrier()

    def issue_gather(c, b):
        pltpu.async_copy(q_hbm.at[dst_all.at[c]], qd.at[b], gq[b])
        pltpu.async_copy(k_hbm.at[src_all.at[c]], ks.at[b], gk[b])

    def wait_gather(c, b):
        pltpu.make_async_copy(q_hbm.at[dst_all.at[c]], qd.at[b], gq[b]).wait()
        pltpu.make_async_copy(k_hbm.at[src_all.at[c]], ks.at[b], gk[b]).wait()

    def issue_writes(c, b4):
        off = base + c * _CA
        pltpu.async_copy(exv.at[b4], ex_hbm.at[pl.ds(off, _CA)], ws[b4])
        pltpu.async_copy(exv.at[b4], den_sh.at[dst_all.at[c]], wd[b4],
                         add=True)

    def wait_writes(c, b4):
        off = base + c * _CA
        pltpu.make_async_copy(exv.at[b4], ex_hbm.at[pl.ds(off, _CA)],
                              ws[b4]).wait()
        pltpu.make_async_copy(exv.at[b4], den_sh.at[dst_all.at[c]],
                              wd[b4]).wait()

    issue_gather(0, 0)
    issue_gather(1, 1)

    @pl.loop(0, nca, step=4)
    def _(c0):
        for db in range(4):
            c = c0 + db
            b2 = db % 2
            b4 = db
            wait_gather(c, b2)

            @pl.loop(0, _CA, step=16)
            def _(g):
                for e in range(16):
                    acc = (qd[b2, g + e, pl.ds(0, 16)]
                           * ks[b2, g + e, pl.ds(0, 16)])
                    for j in range(1, 16):
                        acc = acc + (qd[b2, g + e, pl.ds(16 * j, 16)]
                                     * ks[b2, g + e, pl.ds(16 * j, 16)])
                    accb[e, :] = acc
                tot = plsc.load_gather(accb, [lanes, jnp.zeros(16, jnp.int32)])
                for j in range(1, 16):
                    tot = tot + plsc.load_gather(
                        accb, [lanes, jnp.full(16, j, jnp.int32)])
                a16 = jnp.exp(tot * (1.0 / 16.0))
                a16 = jnp.where(base + c * _CA + g + lanes < E, a16, 0.0)
                exv[b4, pl.ds(g, 16)] = a16

            issue_writes(c, b4)

            @pl.when(c >= 2)
            def _():
                wait_writes(c - 2, (db + 2) % 4)

            @pl.when(c + 2 < nca)
            def _():
                issue_gather(c + 2, b2)

    wait_writes(nca - 2, 2)
    wait_writes(nca - 1, 3)

    plsc.subcore_barrier()

    @pl.when(sid == 0)
    def _():
        pltpu.sync_copy(den_sh, den_hbm.at[cid])


@functools.partial(
    pl.kernel,
    mesh=_mesh,
    compiler_params=_sc_params,
    out_type=jax.ShapeDtypeStruct((2, N, DH), jnp.float32),
    scratch_types=[
        pltpu.VMEM((8, _CB), jnp.int32),
        pltpu.VMEM((8, _CB), jnp.int32),
        pltpu.VMEM((8, _CB), jnp.float32),
        pltpu.VMEM((4, _CB, DH), jnp.float32),
        pltpu.VMEM_SHARED((N, DH), jnp.float32),
        pltpu.SemaphoreType.DMA,
        pltpu.SemaphoreType.DMA,
        pltpu.SemaphoreType.DMA,
        pltpu.SemaphoreType.DMA,
        pltpu.SemaphoreType.DMA,
        pltpu.SemaphoreType.DMA,
        pltpu.SemaphoreType.DMA,
        pltpu.SemaphoreType.DMA,
        pltpu.SemaphoreType.DMA,
        pltpu.SemaphoreType.DMA,
        pltpu.SemaphoreType.DMA,
        pltpu.SemaphoreType.DMA,
        pltpu.SemaphoreType.DMA,
        pltpu.SemaphoreType.DMA,
        pltpu.SemaphoreType.DMA,
        pltpu.SemaphoreType.DMA,
    ],
)
def _edge_aggregate(vcat_hbm, src2_hbm, dstb_hbm, exb_hbm, znd_hbm, out_hbm,
                    srcc, dstc, exc, rows, out_sh,
                    gs0, gs1, gs2, gs3, ss0, ss1, ss2, ss3,
                    sm0, sm1, sm2, sm3, sm4, sm5, sm6, sm7):
    cid = lax.axis_index("c")
    sid = lax.axis_index("s")
    gs = (gs0, gs1, gs2, gs3)
    ss = (ss0, ss1, ss2, ss3)
    sm = (sm0, sm1, sm2, sm3, sm4, sm5, sm6, sm7)

    @pl.when(sid == 0)
    def _():
        pltpu.sync_copy(znd_hbm, out_sh)

    plsc.subcore_barrier()

    def issue_small(c, b8):
        pltpu.async_copy(src2_hbm.at[cid, sid, c], srcc.at[b8], sm[b8])
        pltpu.async_copy(dstb_hbm.at[sid, c], dstc.at[b8], sm[b8])
        pltpu.async_copy(exb_hbm.at[sid, c], exc.at[b8], sm[b8])

    def wait_small(c, b8):
        pltpu.make_async_copy(src2_hbm.at[cid, sid, c], srcc.at[b8],
                              sm[b8]).wait()
        pltpu.make_async_copy(dstb_hbm.at[sid, c], dstc.at[b8],
                              sm[b8]).wait()
        pltpu.make_async_copy(exb_hbm.at[sid, c], exc.at[b8],
                              sm[b8]).wait()

    def issue_gather(c, b4, b8):
        pltpu.async_copy(vcat_hbm.at[srcc.at[b8]], rows.at[b4], gs[b4])

    def wait_gather(c, b4, b8):
        pltpu.make_async_copy(vcat_hbm.at[srcc.at[b8]], rows.at[b4],
                              gs[b4]).wait()

    def issue_scatter(c, b4, b8):
        pltpu.async_copy(rows.at[b4], out_sh.at[dstc.at[b8]], ss[b4],
                         add=True)

    def wait_scatter(c, b4, b8):
        pltpu.make_async_copy(rows.at[b4], out_sh.at[dstc.at[b8]],
                              ss[b4]).wait()

    for c in range(4):
        issue_small(c, c)
    for c in range(2):
        wait_small(c, c)
        issue_gather(c, c, c)

    @pl.loop(0, _NCB, step=8)
    def _(c0):
        for db in range(8):
            c = c0 + db
            b4 = db % 4
            b8 = db
            wait_gather(c, b4, b8)

            @pl.loop(0, _CB, step=2)
            def _(e):
                for de in range(2):
                    s = plsc.load_gather(
                        exc, [jnp.full(16, b8, jnp.int32),
                              jnp.full(16, e + de, jnp.int32)])
                    for j in range(DH // 16):
                        sl = pl.ds(16 * j, 16)
                        rows[b4, e + de, sl] = rows[b4, e + de, sl] * s

            issue_scatter(c, b4, b8)

            @pl.when(c >= 2)
            def _():
                wait_scatter(c - 2, (db + 2) % 4, (db + 6) % 8)

            @pl.when(c + 2 < _NCB)
            def _():
                wait_small(c + 2, (db + 2) % 8)
                issue_gather(c + 2, (db + 2) % 4, (db + 2) % 8)

            @pl.when(c + 4 < _NCB)
            def _():
                issue_small(c + 4, (db + 4) % 8)

    wait_scatter(_NCB - 2, (_NCB - 2) % 4, (_NCB - 2) % 8)
    wait_scatter(_NCB - 1, (_NCB - 1) % 4, (_NCB - 1) % 8)

    plsc.subcore_barrier()

    @pl.when(sid == 0)
    def _():
        pltpu.sync_copy(out_sh, out_hbm.at[cid])


_NB = 10


def _epilogue_body(outp_ref, den_ref, skip_ref, wkp_ref, bkp_ref,
                   wvp_ref, bvp_ref, qv_ref, wop_ref, bop_ref,
                   h_ref, s_ref, y_ref, m_ref, l_ref,
                   m_s, l_s, num_s):
    i = pl.program_id(0)
    out = jnp.concatenate([outp_ref[0], outp_ref[1]], axis=1)
    den = jnp.maximum(den_ref[0] + den_ref[1], 1e-16)
    z = out / den + skip_ref[...]
    hb = 0.5 * z * (1.0 + lax.erf(z * 0.7071067811865476))
    h_ref[...] = hb

    kp = jnp.dot(hb, wkp_ref[...], preferred_element_type=jnp.float32,
                 precision=jax.lax.Precision.HIGHEST) + bkp_ref[...]
    sb = jnp.dot(kp, qv_ref[...].T, preferred_element_type=jnp.float32,
                 precision=jax.lax.Precision.HIGHEST) * (1.0 / 16.0)
    s_ref[...] = sb
    vp = jnp.dot(hb, wvp_ref[...], preferred_element_type=jnp.float32,
                 precision=jax.lax.Precision.HIGHEST) + bvp_ref[...]

    @pl.when(i == 0)
    def _():
        m_s[0, 0] = -jnp.inf
        l_s[0, 0] = 0.0
        num_s[...] = jnp.zeros_like(num_s)

    m_old = m_s[0, 0]
    m_new = jnp.maximum(m_old, jnp.max(sb))
    corr = jnp.exp(m_old - m_new)
    w = jnp.exp(sb - m_new)
    l_s[0, 0] = l_s[0, 0] * corr + jnp.sum(w)
    num_s[...] = num_s[...] * corr + jnp.dot(
        w.T, vp, preferred_element_type=jnp.float32,
        precision=jax.lax.Precision.HIGHEST)
    m_s[0, 0] = m_new

    @pl.when(i == _NB - 1)
    def _():
        pooled = num_s[...] / l_s[0, 0]
        y_ref[...] = jnp.dot(pooled, wop_ref[...],
                             preferred_element_type=jnp.float32,
                             precision=jax.lax.Precision.HIGHEST) + bop_ref[...]
        m_ref[...] = jnp.full((1, 1), m_s[0, 0], jnp.float32)
        l_ref[...] = jnp.full((1, 1), l_s[0, 0], jnp.float32)


def _epilogue(outp, den2, skip, wkp, bkp, wvp, bvp, qv, wop, bop):
    r = N // _NB
    return pl.pallas_call(
        _epilogue_body,
        grid=(_NB,),
        in_specs=[
            pl.BlockSpec((2, r, DH), lambda i: (0, i, 0)),
            pl.BlockSpec((2, r, 1), lambda i: (0, i, 0)),
            pl.BlockSpec((r, D), lambda i: (i, 0)),
            pl.BlockSpec((D, D), lambda i: (0, 0)),
            pl.BlockSpec((1, D), lambda i: (0, 0)),
            pl.BlockSpec((D, D), lambda i: (0, 0)),
            pl.BlockSpec((1, D), lambda i: (0, 0)),
            pl.BlockSpec((1, D), lambda i: (0, 0)),
            pl.BlockSpec((D, D), lambda i: (0, 0)),
            pl.BlockSpec((1, D), lambda i: (0, 0)),
        ],
        out_specs=[
            pl.BlockSpec((r, D), lambda i: (i, 0)),
            pl.BlockSpec((r, 1), lambda i: (i, 0)),
            pl.BlockSpec((1, D), lambda i: (0, 0)),
            pl.BlockSpec((1, 1), lambda i: (0, 0)),
            pl.BlockSpec((1, 1), lambda i: (0, 0)),
        ],
        out_shape=[
            jax.ShapeDtypeStruct((N, D), jnp.float32),
            jax.ShapeDtypeStruct((N, 1), jnp.float32),
            jax.ShapeDtypeStruct((1, D), jnp.float32),
            jax.ShapeDtypeStruct((1, 1), jnp.float32),
            jax.ShapeDtypeStruct((1, 1), jnp.float32),
        ],
        scratch_shapes=[
            pltpu.SMEM((1, 1), jnp.float32),
            pltpu.SMEM((1, 1), jnp.float32),
            pltpu.VMEM((1, D), jnp.float32),
        ],
    )(outp, den2, skip, wkp, bkp, wvp, bvp, qv, wop, bop)


def _attn_body(s_ref, m_ref, l_ref, a_ref):
    a_ref[...] = jnp.exp(s_ref[...] - m_ref[0, 0]) / l_ref[0, 0]


def _attn_finalize(scores, m, l):
    r = N // _NB
    return pl.pallas_call(
        _attn_body,
        grid=(_NB,),
        in_specs=[
            pl.BlockSpec((r, 1), lambda i: (i, 0)),
            pl.BlockSpec((1, 1), lambda i: (0, 0)),
            pl.BlockSpec((1, 1), lambda i: (0, 0)),
        ],
        out_specs=pl.BlockSpec((r, 1), lambda i: (i, 0)),
        out_shape=jax.ShapeDtypeStruct((N, 1), jnp.float32),
    )(scores, m, l)


def kernel(x, edge_index, label, params):
    p = params
    wqk = jnp.concatenate([p['Wq'], p['Wk']], axis=1)
    bqk = jnp.concatenate([p['bq'], p['bk']])[None, :]
    wvs = jnp.concatenate([p['Wv'], p['Ws']], axis=1)
    bvs = jnp.concatenate([p['bv'], p['bs']])[None, :]
    qk = _projections(x, wqk, bqk)
    q = qk[:, 0:D]
    k = qk[:, D:2 * D]
    v2, skip = _vprojections(x, wvs, bvs)
    vcat = v2.reshape(2 * N, DH)

    src = jnp.pad(edge_index[0], (0, _EPAD - E))
    dst = jnp.pad(edge_index[1], (0, _EPAD - E))
    src3 = src.reshape(_EPAD // _CA, _CA)
    dst3 = dst.reshape(_EPAD // _CA, _CA)
    srcb = src.reshape(16, _NCB, _CB)
    dstb = dst.reshape(16, _NCB, _CB)
    src2 = jnp.stack([srcb, srcb + N])
    zn = jnp.zeros((N,), jnp.float32)
    znd = jnp.zeros((N, DH), jnp.float32)

    ex, den2 = _edge_softmax(q, k, src3, dst3, zn)
    exb = ex.reshape(16, _NCB, _CB)
    outp = _edge_aggregate(vcat, src2, dstb, exb, znd)

    qv = p['seed'] @ p['Wqp'] + p['bqp']
    h, scores, y, m, l = _epilogue(
        outp, den2[:, :, None], skip, p['Wkp'], p['bkp'][None, :],
        p['Wvp'], p['bvp'][None, :], qv, p['Wop'], p['bop'][None, :])
    A = _attn_finalize(scores, m, l)[:, 0]
    return (y, A, h)
